# baseline (speedup 1.0000x reference)
import functools

import jax
import jax.numpy as jnp
from jax import lax
from jax.experimental import pallas as pl
from jax.experimental.pallas import tpu as pltpu
from jax.experimental.pallas import tpu_sc as plsc

_B, _L, _D = 4096, 50, 128
_NC, _NS = 2, 16
_NW = _NC * _NS
_BPW = _B // _NW
_CH = 8
_LANES = 16


_NBUF = 4


def _pool_body(iw, ib, it, ew, ebi, etri, out,
               idx_w, idx_b, idx_t, rows, acc, sems):
    wid = lax.axis_index("s") * _NC + lax.axis_index("c")
    base = wid * _BPW

    pltpu.sync_copy(iw.at[pl.ds(base, _BPW)], idx_w)
    pltpu.sync_copy(ib.at[pl.ds(base, _BPW)], idx_b)
    pltpu.sync_copy(it.at[pl.ds(base, _BPW)], idx_t)

    def fire(bi, k):
        pltpu.async_copy(ew.at[idx_w.at[bi]], rows.at[k, pl.ds(0, _L)], sems[k])
        pltpu.async_copy(ebi.at[idx_b.at[bi]], rows.at[k, pl.ds(_L, _L)], sems[k])
        pltpu.async_copy(etri.at[idx_t.at[bi]], rows.at[k, pl.ds(2 * _L, _L)], sems[k])

    def drain(k):
        pltpu.make_async_copy(ew.at[idx_w.at[0]], rows.at[k, pl.ds(0, _L)], sems[k]).wait()
        pltpu.make_async_copy(ebi.at[idx_b.at[0]], rows.at[k, pl.ds(_L, _L)], sems[k]).wait()
        pltpu.make_async_copy(etri.at[idx_t.at[0]], rows.at[k, pl.ds(2 * _L, _L)], sems[k]).wait()

    ndd = _D // _LANES
    nacc = 5

    def accum(b, k):
        for t in range(3):
            for d in range(ndd):
                sl = pl.ds(d * _LANES, _LANES)
                parts = [rows[k, t * _L + a, sl] for a in range(nacc)]
                for j in range(nacc, _L):
                    parts[j % nacc] = parts[j % nacc] + rows[k, t * _L + j, sl]
                s = (parts[0] + parts[1]) + (parts[2] + parts[3]) + parts[4]
                acc[b, pl.ds(t * _D + d * _LANES, _LANES)] = s

    for k in range(_NBUF - 1):
        fire(k, k)
    for k in range(_NBUF - 1):
        drain(k)

    def chunk(c, _):
        cbase = c * _CH

        def body(i, _):
            b0 = cbase + _NBUF * i
            r = _NBUF * i
            for k in range(_NBUF):
                accum(r + k, 0)
            return ()

        lax.fori_loop(0, _CH // _NBUF, body, ())
        pltpu.sync_copy(acc, out.at[pl.ds(base + cbase, _CH)])
        return ()

    lax.fori_loop(0, _BPW // _CH, chunk, ())


@functools.partial(
    pl.kernel,
    out_type=jax.ShapeDtypeStruct((_B, 3 * _D), jnp.float32),
    mesh=plsc.VectorSubcoreMesh(
        core_axis_name="c", subcore_axis_name="s",
        num_cores=_NC, num_subcores=_NS),
    scratch_types=[
        pltpu.VMEM((_BPW, _L), jnp.int32),
        pltpu.VMEM((_BPW, _L), jnp.int32),
        pltpu.VMEM((_BPW, _L), jnp.int32),
        pltpu.VMEM((_NBUF, 3 * _L, _D), jnp.float32),
        pltpu.VMEM((_CH, 3 * _D), jnp.float32),
    ] + [pltpu.SemaphoreType.DMA] * _NBUF,
)
def _pooled_embed(iw, ib, it, ew, ebi, etri, out,
                  idx_w, idx_b, idx_t, rows, acc, *sems):
    _pool_body(iw, ib, it, ew, ebi, etri, out,
               idx_w, idx_b, idx_t, rows, acc, list(sems))


def _mlp_kernel(x_ref, w1_ref, b1_ref, w2_ref, b2_ref, o_ref):
    x = x_ref[...] * (1.0 / _L)
    h = jnp.dot(x, w1_ref[...], preferred_element_type=jnp.float32)
    h = jnp.maximum(h + b1_ref[...], 0.0)
    o_ref[...] = jnp.dot(h, w2_ref[...],
                         preferred_element_type=jnp.float32) + b2_ref[...]


def kernel(input_word, input_bigram, input_trigram,
           emb_word, emb_bi, emb_tri, W1, b1, W2, b2):
    pooled = _pooled_embed(input_word, input_bigram, input_trigram,
                           emb_word, emb_bi, emb_tri)
    H = W1.shape[1]
    C = W2.shape[1]
    bm = 512
    out = pl.pallas_call(
        _mlp_kernel,
        grid=(_B // bm,),
        in_specs=[
            pl.BlockSpec((bm, 3 * _D), lambda i: (i, 0)),
            pl.BlockSpec((3 * _D, H), lambda i: (0, 0)),
            pl.BlockSpec((H,), lambda i: (0,)),
            pl.BlockSpec((H, C), lambda i: (0, 0)),
            pl.BlockSpec((C,), lambda i: (0,)),
        ],
        out_specs=pl.BlockSpec((bm, C), lambda i: (i, 0)),
        out_shape=jax.ShapeDtypeStruct((_B, C), jnp.float32),
    )(pooled, W1, b1, W2, b2)
    return out


# speedup vs baseline: 1.2893x; 1.2893x over previous
import functools

import jax
import jax.numpy as jnp
from jax import lax
from jax.experimental import pallas as pl
from jax.experimental.pallas import tpu as pltpu
from jax.experimental.pallas import tpu_sc as plsc

_B, _L, _D = 4096, 50, 128
_NC, _NS = 2, 16
_NW = _NC * _NS
_BPW = _B // _NW
_CH = 16
_LANES = 16
_SLOTS = _NS * _BPW


def _pool_body(iw, ib, it, ew, ebi, etri, sidx, out,
               idx_w, idx_b, idx_t, rows, idxc, shacc,
               sem_g0, sem_g1, sem_s0, sem_s1):
    s = lax.axis_index("s")
    wid = s * _NC + lax.axis_index("c")
    base = wid * _BPW
    gsem = (sem_g0, sem_g1)
    ssem = (sem_s0, sem_s1)

    for j in range(_BPW):
        for d in range(_D // _LANES):
            rows[0, j, pl.ds(d * _LANES, _LANES)] = jnp.zeros(
                (_LANES,), jnp.float32)
    for t in range(3):
        pltpu.sync_copy(rows.at[0, pl.ds(0, _BPW)],
                        shacc.at[pl.ds(t * _SLOTS + s * _BPW, _BPW)])

    def fire(bi, k):
        pltpu.async_copy(ew.at[idx_w.at[bi]], rows.at[k, pl.ds(0, _L)],
                         gsem[k])
        pltpu.async_copy(ebi.at[idx_b.at[bi]], rows.at[k, pl.ds(_L, _L)],
                         gsem[k])
        pltpu.async_copy(etri.at[idx_t.at[bi]], rows.at[k, pl.ds(2 * _L, _L)],
                         gsem[k])

    def drain(k):
        pltpu.make_async_copy(ew.at[idx_w.at[0]], rows.at[k, pl.ds(0, _L)],
                              gsem[k]).wait()
        pltpu.make_async_copy(ebi.at[idx_b.at[0]], rows.at[k, pl.ds(_L, _L)],
                              gsem[k]).wait()
        pltpu.make_async_copy(etri.at[idx_t.at[0]], rows.at[k, pl.ds(2 * _L, _L)],
                              gsem[k]).wait()

    def scat(r, k):
        for t in range(3):
            pltpu.async_copy(rows.at[k, pl.ds(t * _L, _L)],
                             shacc.at[idxc.at[r, t]], ssem[k], add=True)

    def sdrain(r, k):
        for t in range(3):
            pltpu.make_async_copy(rows.at[k, pl.ds(t * _L, _L)],
                                  shacc.at[idxc.at[r, t]], ssem[k]).wait()

    def chunk(c, _):
        cbase = c * _CH
        pltpu.sync_copy(iw.at[pl.ds(base + cbase, _CH)], idx_w)
        pltpu.sync_copy(ib.at[pl.ds(base + cbase, _CH)], idx_b)
        pltpu.sync_copy(it.at[pl.ds(base + cbase, _CH)], idx_t)
        pltpu.sync_copy(sidx.at[s, pl.ds(cbase, _CH)], idxc)
        fire(0, 0)
        fire(1, 1)

        def body(i, _):
            for k in range(2):
                r = 2 * i + k
                drain(k)
                scat(r, k)
                sdrain(r, k)
                fire(jnp.minimum(r + 2, _CH - 1), k)
            return ()

        lax.fori_loop(0, _CH // 2, body, ())
        drain(0)
        drain(1)
        return ()

    lax.fori_loop(0, _BPW // _CH, chunk, ())

    for t in range(3):
        pltpu.sync_copy(shacc.at[pl.ds(t * _SLOTS + s * _BPW, _BPW)],
                        out.at[pl.ds(base, _BPW), pl.ds(t * _D, _D)])


@functools.partial(
    pl.kernel,
    out_type=jax.ShapeDtypeStruct((_B, 3 * _D), jnp.float32),
    mesh=plsc.VectorSubcoreMesh(
        core_axis_name="c", subcore_axis_name="s",
        num_cores=_NC, num_subcores=_NS),
    scratch_types=[
        pltpu.VMEM((_CH, _L), jnp.int32),
        pltpu.VMEM((_CH, _L), jnp.int32),
        pltpu.VMEM((_CH, _L), jnp.int32),
        pltpu.VMEM((2, 3 * _L, _D), jnp.float32),
        pltpu.VMEM((_CH, 3, _L), jnp.int32),
        pltpu.VMEM_SHARED((3 * _SLOTS, _D), jnp.float32),
        pltpu.SemaphoreType.DMA,
        pltpu.SemaphoreType.DMA,
        pltpu.SemaphoreType.DMA,
        pltpu.SemaphoreType.DMA,
    ],
)
def _pooled_embed(iw, ib, it, ew, ebi, etri, sidx, out,
                  idx_w, idx_b, idx_t, rows, idxc, shacc,
                  sem_g0, sem_g1, sem_s0, sem_s1):
    _pool_body(iw, ib, it, ew, ebi, etri, sidx, out,
               idx_w, idx_b, idx_t, rows, idxc, shacc,
               sem_g0, sem_g1, sem_s0, sem_s1)


def _mlp_kernel(x_ref, w1_ref, b1_ref, w2_ref, b2_ref, o_ref):
    x = x_ref[...] * (1.0 / _L)
    h = jnp.dot(x, w1_ref[...], preferred_element_type=jnp.float32)
    h = jnp.maximum(h + b1_ref[...], 0.0)
    o_ref[...] = jnp.dot(h, w2_ref[...],
                         preferred_element_type=jnp.float32) + b2_ref[...]


def kernel(input_word, input_bigram, input_trigram,
           emb_word, emb_bi, emb_tri, W1, b1, W2, b2):
    s_ax = jnp.arange(_NS, dtype=jnp.int32)[:, None, None, None]
    r_ax = jnp.arange(_BPW, dtype=jnp.int32)[None, :, None, None]
    t_ax = jnp.arange(3, dtype=jnp.int32)[None, None, :, None]
    sidx = jnp.broadcast_to(
        t_ax * _SLOTS + s_ax * _BPW + r_ax,
        (_NS, _BPW, 3, _L)).astype(jnp.int32)
    pooled = _pooled_embed(input_word, input_bigram, input_trigram,
                           emb_word, emb_bi, emb_tri, sidx)
    H = W1.shape[1]
    C = W2.shape[1]
    bm = 512
    out = pl.pallas_call(
        _mlp_kernel,
        grid=(_B // bm,),
        in_specs=[
            pl.BlockSpec((bm, 3 * _D), lambda i: (i, 0)),
            pl.BlockSpec((3 * _D, H), lambda i: (0, 0)),
            pl.BlockSpec((H,), lambda i: (0,)),
            pl.BlockSpec((H, C), lambda i: (0, 0)),
            pl.BlockSpec((C,), lambda i: (0,)),
        ],
        out_specs=pl.BlockSpec((bm, C), lambda i: (i, 0)),
        out_shape=jax.ShapeDtypeStruct((_B, C), jnp.float32),
    )(pooled, W1, b1, W2, b2)
    return out


# speedup vs baseline: 1.3179x; 1.0222x over previous
import functools

import jax
import jax.numpy as jnp
from jax import lax
from jax.experimental import pallas as pl
from jax.experimental.pallas import tpu as pltpu
from jax.experimental.pallas import tpu_sc as plsc

_B, _L, _D = 4096, 50, 128
_NC, _NS = 2, 16
_NW = _NC * _NS
_BPW = _B // _NW
_CH = 16
_LANES = 16
_SLOTS = _NS * _BPW


_NBUF = 6


def _pool_body(iw, ib, it, ew, ebi, etri, sidx, out,
               idx_w, idx_b, idx_t, rows, idxc, shacc, *sems):
    s = lax.axis_index("s")
    wid = s * _NC + lax.axis_index("c")
    base = wid * _BPW
    gsem = sems[:_NBUF]
    ssem = sems[_NBUF:]
    tables = (ew, ebi, etri)
    idxs = (idx_w, idx_b, idx_t)

    for j in range(3 * _L):
        for d in range(_D // _LANES):
            rows[j // _L, j % _L, pl.ds(d * _LANES, _LANES)] = jnp.zeros(
                (_LANES,), jnp.float32)
    for t in range(3):
        reg = t * _SLOTS + s * _BPW
        pltpu.sync_copy(rows.at[0], shacc.at[pl.ds(reg, _L)])
        pltpu.sync_copy(rows.at[1], shacc.at[pl.ds(reg + _L, _L)])
        pltpu.sync_copy(rows.at[2, pl.ds(0, _BPW - 2 * _L)],
                        shacc.at[pl.ds(reg + 2 * _L, _BPW - 2 * _L)])

    def gfire(lr, t, k):
        pltpu.async_copy(tables[t].at[idxs[t].at[lr]], rows.at[k], gsem[k])

    def gdrain(k):
        pltpu.make_async_copy(ew.at[idx_w.at[0]], rows.at[k], gsem[k]).wait()

    def sfire(lr, t, k):
        pltpu.async_copy(rows.at[k], shacc.at[idxc.at[lr, t]],
                         ssem[k], add=True)

    def swait(k):
        pltpu.make_async_copy(rows.at[k], shacc.at[idxc.at[0, 0]],
                              ssem[k]).wait()

    def step(i, k, first, last):
        r = 2 * i + k // 3
        gdrain(k)
        sfire(r, k % 3, k)
        if not last:
            km = (k + 3) % _NBUF
            if not first:
                swait(km)
            gfire(2 * i + (k + 3) // 3, (k + 3) % 3, km)

    def chunk(c, _):
        cbase = c * _CH
        pltpu.sync_copy(iw.at[pl.ds(base + cbase, _CH)], idx_w)
        pltpu.sync_copy(ib.at[pl.ds(base + cbase, _CH)], idx_b)
        pltpu.sync_copy(it.at[pl.ds(base + cbase, _CH)], idx_t)
        pltpu.sync_copy(sidx.at[s, pl.ds(cbase, _CH)], idxc)
        for t in range(3):
            gfire(0, t, t)

        for k in range(_NBUF):
            step(0, k, first=(k < 3), last=False)

        def body(i, _):
            for k in range(_NBUF):
                step(i, k, first=False, last=False)
            return ()

        lax.fori_loop(1, _CH // 2 - 1, body, ())

        for k in range(_NBUF):
            step(_CH // 2 - 1, k, first=False, last=(k >= 3))
        for k in range(_NBUF):
            swait(k)
        return ()

    lax.fori_loop(0, _BPW // _CH, chunk, ())

    for t in range(3):
        pltpu.sync_copy(shacc.at[pl.ds(t * _SLOTS + s * _BPW, _BPW)],
                        out.at[pl.ds(base, _BPW), pl.ds(t * _D, _D)])


@functools.partial(
    pl.kernel,
    out_type=jax.ShapeDtypeStruct((_B, 3 * _D), jnp.float32),
    mesh=plsc.VectorSubcoreMesh(
        core_axis_name="c", subcore_axis_name="s",
        num_cores=_NC, num_subcores=_NS),
    scratch_types=[
        pltpu.VMEM((_CH, _L), jnp.int32),
        pltpu.VMEM((_CH, _L), jnp.int32),
        pltpu.VMEM((_CH, _L), jnp.int32),
        pltpu.VMEM((_NBUF, _L, _D), jnp.float32),
        pltpu.VMEM((_CH, 3, _L), jnp.int32),
        pltpu.VMEM_SHARED((3 * _SLOTS, _D), jnp.float32),
    ] + [pltpu.SemaphoreType.DMA] * (2 * _NBUF),
)
def _pooled_embed(iw, ib, it, ew, ebi, etri, sidx, out,
                  idx_w, idx_b, idx_t, rows, idxc, shacc, *sems):
    _pool_body(iw, ib, it, ew, ebi, etri, sidx, out,
               idx_w, idx_b, idx_t, rows, idxc, shacc, *sems)


def _mlp_kernel(x_ref, w1_ref, b1_ref, w2_ref, b2_ref, o_ref):
    x = x_ref[...] * (1.0 / _L)
    h = jnp.dot(x, w1_ref[...], preferred_element_type=jnp.float32)
    h = jnp.maximum(h + b1_ref[...], 0.0)
    o_ref[...] = jnp.dot(h, w2_ref[...],
                         preferred_element_type=jnp.float32) + b2_ref[...]


def kernel(input_word, input_bigram, input_trigram,
           emb_word, emb_bi, emb_tri, W1, b1, W2, b2):
    s_ax = jnp.arange(_NS, dtype=jnp.int32)[:, None, None, None]
    r_ax = jnp.arange(_BPW, dtype=jnp.int32)[None, :, None, None]
    t_ax = jnp.arange(3, dtype=jnp.int32)[None, None, :, None]
    sidx = jnp.broadcast_to(
        t_ax * _SLOTS + s_ax * _BPW + r_ax,
        (_NS, _BPW, 3, _L)).astype(jnp.int32)
    pooled = _pooled_embed(input_word, input_bigram, input_trigram,
                           emb_word, emb_bi, emb_tri, sidx)
    H = W1.shape[1]
    C = W2.shape[1]
    bm = 512
    out = pl.pallas_call(
        _mlp_kernel,
        grid=(_B // bm,),
        in_specs=[
            pl.BlockSpec((bm, 3 * _D), lambda i: (i, 0)),
            pl.BlockSpec((3 * _D, H), lambda i: (0, 0)),
            pl.BlockSpec((H,), lambda i: (0,)),
            pl.BlockSpec((H, C), lambda i: (0, 0)),
            pl.BlockSpec((C,), lambda i: (0,)),
        ],
        out_specs=pl.BlockSpec((bm, C), lambda i: (i, 0)),
        out_shape=jax.ShapeDtypeStruct((_B, C), jnp.float32),
    )(pooled, W1, b1, W2, b2)
    return out


# speedup vs baseline: 1.3345x; 1.0126x over previous
import functools

import jax
import jax.numpy as jnp
from jax import lax
from jax.experimental import pallas as pl
from jax.experimental.pallas import tpu as pltpu
from jax.experimental.pallas import tpu_sc as plsc

_B, _L, _D = 4096, 50, 128
_NC, _NS = 2, 16
_NW = _NC * _NS
_BPW = _B // _NW
_CH = 16
_LANES = 16
_SLOTS = _NS * _BPW


_NBUF = 6


def _pool_body(iw, ib, it, ew, ebi, etri, sidx, out,
               idx_w, idx_b, idx_t, rows, idxc, shacc, acc, *sems):
    s = lax.axis_index("s")
    wid = s * _NC + lax.axis_index("c")
    base = wid * _BPW
    gsem = sems[:_NBUF]
    ssem = sems[_NBUF:]
    tables = (ew, ebi, etri)
    idxs = (idx_w, idx_b, idx_t)

    for j in range(3 * _L):
        for d in range(_D // _LANES):
            rows[j // _L, j % _L, pl.ds(d * _LANES, _LANES)] = jnp.zeros(
                (_LANES,), jnp.float32)
    for t in (1, 2):
        reg = t * _SLOTS + s * _BPW
        pltpu.sync_copy(rows.at[0], shacc.at[pl.ds(reg, _L)])
        pltpu.sync_copy(rows.at[1], shacc.at[pl.ds(reg + _L, _L)])
        pltpu.sync_copy(rows.at[2, pl.ds(0, _BPW - 2 * _L)],
                        shacc.at[pl.ds(reg + 2 * _L, _BPW - 2 * _L)])

    def gfire(lr, t, k):
        pltpu.async_copy(tables[t].at[idxs[t].at[lr]], rows.at[k], gsem[k])

    def gdrain(k):
        pltpu.make_async_copy(ew.at[idx_w.at[0]], rows.at[k], gsem[k]).wait()

    def sfire(lr, t, k):
        pltpu.async_copy(rows.at[k], shacc.at[idxc.at[lr, t]],
                         ssem[k], add=True)

    def swait(k):
        pltpu.make_async_copy(rows.at[k], shacc.at[idxc.at[0, 0]],
                              ssem[k]).wait()

    ndd = _D // _LANES
    nacc = 5

    def accum(r, k):
        for d in range(ndd):
            sl = pl.ds(d * _LANES, _LANES)
            parts = [rows[k, a, sl] for a in range(nacc)]
            for j in range(nacc, _L):
                parts[j % nacc] = parts[j % nacc] + rows[k, j, sl]
            acc[r, sl] = (parts[0] + parts[1]) + (parts[2] + parts[3]) \
                + parts[4]

    def step(i, k, first, last):
        r = 2 * i + k // 3
        t = k % 3
        gdrain(k)
        km = (k + 3) % _NBUF
        if t == 0:
            if not last:
                gfire(2 * i + (k + 3) // 3, 0, km)
            accum(r, k)
        else:
            sfire(r, t, k)
            if not last:
                if not first:
                    swait(km)
                gfire(2 * i + (k + 3) // 3, t, km)

    def chunk(c, _):
        cbase = c * _CH
        pltpu.sync_copy(iw.at[pl.ds(base + cbase, _CH)], idx_w)
        pltpu.sync_copy(ib.at[pl.ds(base + cbase, _CH)], idx_b)
        pltpu.sync_copy(it.at[pl.ds(base + cbase, _CH)], idx_t)
        pltpu.sync_copy(sidx.at[s, pl.ds(cbase, _CH)], idxc)
        for t in range(3):
            gfire(0, t, t)

        for k in range(_NBUF):
            step(0, k, first=(k < 3), last=False)

        def body(i, _):
            for k in range(_NBUF):
                step(i, k, first=False, last=False)
            return ()

        lax.fori_loop(1, _CH // 2 - 1, body, ())

        for k in range(_NBUF):
            step(_CH // 2 - 1, k, first=False, last=(k >= 3))
        for k in (1, 2, 4, 5):
            swait(k)
        pltpu.sync_copy(acc, out.at[pl.ds(base + cbase, _CH), pl.ds(0, _D)])
        return ()

    lax.fori_loop(0, _BPW // _CH, chunk, ())

    for t in (1, 2):
        pltpu.sync_copy(shacc.at[pl.ds(t * _SLOTS + s * _BPW, _BPW)],
                        out.at[pl.ds(base, _BPW), pl.ds(t * _D, _D)])


@functools.partial(
    pl.kernel,
    out_type=jax.ShapeDtypeStruct((_B, 3 * _D), jnp.float32),
    mesh=plsc.VectorSubcoreMesh(
        core_axis_name="c", subcore_axis_name="s",
        num_cores=_NC, num_subcores=_NS),
    scratch_types=[
        pltpu.VMEM((_CH, _L), jnp.int32),
        pltpu.VMEM((_CH, _L), jnp.int32),
        pltpu.VMEM((_CH, _L), jnp.int32),
        pltpu.VMEM((_NBUF, _L, _D), jnp.float32),
        pltpu.VMEM((_CH, 3, _L), jnp.int32),
        pltpu.VMEM_SHARED((3 * _SLOTS, _D), jnp.float32),
        pltpu.VMEM((_CH, _D), jnp.float32),
    ] + [pltpu.SemaphoreType.DMA] * (2 * _NBUF),
)
def _pooled_embed(iw, ib, it, ew, ebi, etri, sidx, out,
                  idx_w, idx_b, idx_t, rows, idxc, shacc, acc, *sems):
    _pool_body(iw, ib, it, ew, ebi, etri, sidx, out,
               idx_w, idx_b, idx_t, rows, idxc, shacc, acc, *sems)


def _mlp_kernel(x_ref, w1_ref, b1_ref, w2_ref, b2_ref, o_ref):
    x = x_ref[...] * (1.0 / _L)
    h = jnp.dot(x, w1_ref[...], preferred_element_type=jnp.float32)
    h = jnp.maximum(h + b1_ref[...], 0.0)
    o_ref[...] = jnp.dot(h, w2_ref[...],
                         preferred_element_type=jnp.float32) + b2_ref[...]


def kernel(input_word, input_bigram, input_trigram,
           emb_word, emb_bi, emb_tri, W1, b1, W2, b2):
    s_ax = jnp.arange(_NS, dtype=jnp.int32)[:, None, None, None]
    r_ax = jnp.arange(_BPW, dtype=jnp.int32)[None, :, None, None]
    t_ax = jnp.arange(3, dtype=jnp.int32)[None, None, :, None]
    sidx = jnp.broadcast_to(
        t_ax * _SLOTS + s_ax * _BPW + r_ax,
        (_NS, _BPW, 3, _L)).astype(jnp.int32)
    pooled = _pooled_embed(input_word, input_bigram, input_trigram,
                           emb_word, emb_bi, emb_tri, sidx)
    H = W1.shape[1]
    C = W2.shape[1]
    bm = 512
    out = pl.pallas_call(
        _mlp_kernel,
        grid=(_B // bm,),
        in_specs=[
            pl.BlockSpec((bm, 3 * _D), lambda i: (i, 0)),
            pl.BlockSpec((3 * _D, H), lambda i: (0, 0)),
            pl.BlockSpec((H,), lambda i: (0,)),
            pl.BlockSpec((H, C), lambda i: (0, 0)),
            pl.BlockSpec((C,), lambda i: (0,)),
        ],
        out_specs=pl.BlockSpec((bm, C), lambda i: (i, 0)),
        out_shape=jax.ShapeDtypeStruct((_B, C), jnp.float32),
    )(pooled, W1, b1, W2, b2)
    return out
